# baseline (speedup 1.0000x reference)
import functools

import jax
import jax.numpy as jnp
from jax import lax
from jax.experimental import pallas as pl
from jax.experimental.pallas import tpu as pltpu
from jax.experimental.pallas import tpu_sc as plsc

D = 128
B_TOK = 16384
S = 20
B = B_TOK * S
NC = 2
NS = 16
NW = NC * NS
BPW = B // NW
C = 128
STEPS = BPW // C
K = 2
G = STEPS // K

_mesh = plsc.VectorSubcoreMesh(core_axis_name="c", subcore_axis_name="s")


@functools.partial(
    pl.kernel,
    mesh=_mesh,
    out_type=jax.ShapeDtypeStruct((B, D), jnp.float32),
    scratch_types=[
        pltpu.VMEM((STEPS, C), jnp.int32),
        pltpu.VMEM((2, K, C, D), jnp.float32),
        pltpu.SemaphoreType.DMA,
        pltpu.SemaphoreType.DMA,
    ],
)
def _gather_rows(table_hbm, idx_hbm, out_hbm, idx_v, rows_v, gsem, osem):
    cid = lax.axis_index("c")
    sid = lax.axis_index("s")
    wid = sid * NC + cid
    base = wid * BPW

    pltpu.sync_copy(idx_hbm.at[wid], idx_v)

    for j in range(K):
        pltpu.async_copy(table_hbm.at[idx_v.at[j]], rows_v.at[0, j], gsem)

    def body(g, _):
        h = lax.rem(g, 2)
        hp = 1 - h

        for j in range(K):
            pltpu.make_async_copy(
                table_hbm.at[idx_v.at[0]], rows_v.at[h, j], gsem).wait()

        @pl.when(g > 0)
        def _():
            for j in range(K):
                pltpu.make_async_copy(
                    rows_v.at[hp, j], out_hbm.at[pl.ds(base, C)], osem).wait()

        @pl.when(g + 1 < G)
        def _():
            for j in range(K):
                step = (g + 1) * K + j
                pltpu.async_copy(
                    table_hbm.at[idx_v.at[step]], rows_v.at[hp, j], gsem)

        for j in range(K):
            step = g * K + j
            pltpu.async_copy(
                rows_v.at[h, j], out_hbm.at[pl.ds(base + step * C, C)], osem)
        return 0

    lax.fori_loop(0, G, body, 0)

    for j in range(K):
        pltpu.make_async_copy(
            rows_v.at[0, j], out_hbm.at[pl.ds(base, C)], osem).wait()


def kernel(weight, token_ids):
    idx = token_ids.reshape(-1).astype(jnp.int32).reshape(NW, STEPS, C)
    out = _gather_rows(weight, idx)
    return out.reshape(B_TOK, S, D)


# speedup vs baseline: 3.2151x; 3.2151x over previous
import functools

import jax
import jax.numpy as jnp
from jax import lax
from jax.experimental import pallas as pl
from jax.experimental.pallas import tpu as pltpu
from jax.experimental.pallas import tpu_sc as plsc

D = 128
B_TOK = 16384
S = 20
B = B_TOK * S
NC = 2
NS = 16
NW = NC * NS
BPW = B // NW
C = 128
STEPS = BPW // C
K = 2
G = STEPS // K

_mesh = plsc.VectorSubcoreMesh(core_axis_name="c", subcore_axis_name="s")


@functools.partial(
    pl.kernel,
    mesh=_mesh,
    out_type=jax.ShapeDtypeStruct((B, D), jnp.float32),
    scratch_types=[
        pltpu.VMEM((STEPS, C), jnp.int32),
        pltpu.VMEM((2, K, C, D), jnp.float32),
        pltpu.SemaphoreType.DMA,
        pltpu.SemaphoreType.DMA,
    ],
)
def _gather_rows(table_hbm, idx_hbm, out_hbm, idx_v, rows_v, gsem, osem):
    cid = lax.axis_index("c")
    sid = lax.axis_index("s")
    wid = sid * NC + cid
    base = wid * BPW

    pltpu.sync_copy(idx_hbm.at[wid], idx_v)

    for j in range(K):
        pltpu.async_copy(table_hbm.at[idx_v.at[j]], rows_v.at[0, j], gsem)

    def body(g, _):
        h = lax.rem(g, 2)
        hp = 1 - h

        for j in range(K):
            pltpu.make_async_copy(
                table_hbm.at[idx_v.at[0]], rows_v.at[h, j], gsem).wait()

        @pl.when(g > 0)
        def _():
            for j in range(K):
                pltpu.make_async_copy(
                    rows_v.at[hp, j], out_hbm.at[pl.ds(base, C)], osem).wait()

        @pl.when(g + 1 < G)
        def _():
            for j in range(K):
                step = (g + 1) * K + j
                pltpu.async_copy(
                    table_hbm.at[idx_v.at[step]], rows_v.at[hp, j], gsem)

        for j in range(K):
            step = g * K + j
            pltpu.async_copy(
                rows_v.at[h, j], out_hbm.at[pl.ds(base + step * C, C)], osem)
        return 0

    lax.fori_loop(0, G, body, 0)

    for j in range(K):
        pltpu.make_async_copy(
            rows_v.at[0, j], out_hbm.at[pl.ds(base, C)], osem).wait()


def kernel(weight, token_ids):
    idx = jnp.transpose(token_ids.astype(jnp.int32)).reshape(NW, STEPS, C)
    out = _gather_rows(weight, idx)
    return jnp.transpose(out.reshape(S, B_TOK, D), (1, 0, 2))


# speedup vs baseline: 3.3416x; 1.0393x over previous
import functools

import jax
import jax.numpy as jnp
from jax import lax
from jax.experimental import pallas as pl
from jax.experimental.pallas import tpu as pltpu
from jax.experimental.pallas import tpu_sc as plsc

D = 128
B_TOK = 16384
S = 20
B = B_TOK * S
NC = 2
NS = 16
NW = NC * NS
BPW = B // NW
C = 128
STEPS = BPW // C
H = 5
PF = 2
M = STEPS // H

_mesh = plsc.VectorSubcoreMesh(core_axis_name="c", subcore_axis_name="s")


@functools.partial(
    pl.kernel,
    mesh=_mesh,
    out_type=jax.ShapeDtypeStruct((B, D), jnp.float32),
    scratch_types=[
        pltpu.VMEM((STEPS, C), jnp.int32),
        pltpu.VMEM((H, C, D), jnp.float32),
        [pltpu.SemaphoreType.DMA] * H,
        [pltpu.SemaphoreType.DMA] * H,
    ],
)
def _gather_rows(table_hbm, idx_hbm, out_hbm, idx_v, rows_v, gsems, osems):
    cid = lax.axis_index("c")
    sid = lax.axis_index("s")
    wid = sid * NC + cid
    base = wid * BPW

    pltpu.sync_copy(idx_hbm.at[wid], idx_v)

    for j in range(PF):
        pltpu.async_copy(table_hbm.at[idx_v.at[j]], rows_v.at[j], gsems[j])

    def body(m, _):
        for h in range(H):
            g = m * H + h

            pltpu.make_async_copy(
                table_hbm.at[idx_v.at[0]], rows_v.at[h], gsems[h]).wait()

            pltpu.async_copy(
                rows_v.at[h], out_hbm.at[pl.ds(base + g * C, C)], osems[h])

            hp = (h + PF) % H
            gp = g + PF

            @pl.when(jnp.logical_and(gp >= H, gp < STEPS))
            def _():
                pltpu.make_async_copy(
                    rows_v.at[hp], out_hbm.at[pl.ds(base, C)], osems[hp]).wait()

            @pl.when(gp < STEPS)
            def _():
                pltpu.async_copy(
                    table_hbm.at[idx_v.at[gp]], rows_v.at[hp], gsems[hp])
        return 0

    lax.fori_loop(0, M, body, 0)

    for g in range(STEPS - H, STEPS):
        h = g % H
        pltpu.make_async_copy(
            rows_v.at[h], out_hbm.at[pl.ds(base, C)], osems[h]).wait()


def kernel(weight, token_ids):
    idx = jnp.transpose(token_ids.astype(jnp.int32)).reshape(NW, STEPS, C)
    out = _gather_rows(weight, idx)
    return jnp.transpose(out.reshape(S, B_TOK, D), (1, 0, 2))
